# baseline (speedup 1.0000x reference)
import functools

import jax
import jax.numpy as jnp
from jax import lax
from jax.experimental import pallas as pl
from jax.experimental.pallas import tpu as pltpu
from jax.experimental.pallas import tpu_sc as plsc

N = 10000
E = 320000
H = 128
C = 16
NC, NS = 2, 16
NW = NC * NS
EPW = E // NW
CHUNK = 80
NCHUNK = EPW // CHUNK
NPAD = 10240
ROWS_PT = NPAD // NS
DEN_PAD = 10240
DEN_PT = DEN_PAD // NS


def _tc_head_body(x_ref, w_ref, att_ref, h_ref, a_ref):
    h = jnp.dot(x_ref[...], w_ref[...], preferred_element_type=jnp.float32)
    h_ref[...] = h
    a_ref[...] = jnp.dot(h, att_ref[...], preferred_element_type=jnp.float32)


def _tc_head(x, w, att2):
    bm = 1000
    return pl.pallas_call(
        _tc_head_body,
        grid=(N // bm,),
        in_specs=[
            pl.BlockSpec((bm, H), lambda i: (i, 0)),
            pl.BlockSpec((H, H), lambda i: (0, 0)),
            pl.BlockSpec((H, 8), lambda i: (0, 0)),
        ],
        out_specs=[
            pl.BlockSpec((bm, H), lambda i: (i, 0)),
            pl.BlockSpec((bm, 8), lambda i: (i, 0)),
        ],
        out_shape=[
            jax.ShapeDtypeStruct((N, H), jnp.float32),
            jax.ShapeDtypeStruct((N, 8), jnp.float32),
        ],
    )(x, w, att2)


def _sc_edge_body(src_hbm, dst_hbm, asrc_hbm, adst_hbm, h_hbm,
                  s_out, den_out,
                  src_v, dst_v, idx_a, idx_b, sga_a, sgb_a, sga_b, sgb_b,
                  row_a, row_b, den_stage, s_sp, den_sp,
                  asem_a, asem_b, rsem_a, rsem_b,
                  ssem_a, ssem_b, dsem_a, dsem_b):
    cid = lax.axis_index("c")
    sid = lax.axis_index("s")
    wid = cid * NS + sid

    zero16 = jnp.zeros((16,), jnp.float32)

    pltpu.sync_copy(src_hbm.at[wid], src_v)
    pltpu.sync_copy(dst_hbm.at[wid], dst_v)

    def zrow(r, _):
        for c8 in range(H // 16):
            row_a[r, pl.ds(c8 * 16, 16)] = zero16
        return 0
    lax.fori_loop(0, CHUNK, zrow, 0)
    def zden(t, _):
        den_stage[pl.ds(t * 16, 16)] = zero16
        return 0
    lax.fori_loop(0, DEN_PT // 16, zden, 0)

    base = sid * ROWS_PT
    def zs(k, _):
        pltpu.sync_copy(row_a, s_sp.at[pl.ds(base + k * CHUNK, CHUNK)])
        return 0
    lax.fori_loop(0, ROWS_PT // CHUNK, zs, 0)

    dbase = sid * DEN_PT
    pltpu.sync_copy(den_stage, den_sp.at[pl.ds(dbase, DEN_PT)])

    plsc.subcore_barrier()

    def zidx(t, _):
        z16 = jnp.zeros((16,), jnp.int32)
        idx_a[0, pl.ds(t * 16, 16)] = z16
        idx_b[0, pl.ds(t * 16, 16)] = z16
        sga_a[pl.ds(t * 16, 16)] = zero16
        sga_b[pl.ds(t * 16, 16)] = zero16
        return 0
    lax.fori_loop(0, CHUNK // 16, zidx, 0)
    def zrowb(r, _):
        for c8 in range(H // 16):
            row_b[r, pl.ds(c8 * 16, 16)] = zero16
        return 0
    lax.fori_loop(0, CHUNK, zrowb, 0)
    pltpu.async_copy(row_a, s_sp.at[idx_a.at[0]], ssem_a, add=True)
    pltpu.async_copy(row_b, s_sp.at[idx_b.at[0]], ssem_b, add=True)
    pltpu.async_copy(sga_a, den_sp.at[idx_a.at[0]], dsem_a, add=True)
    pltpu.async_copy(sga_b, den_sp.at[idx_b.at[0]], dsem_b, add=True)

    def fire(j, sga, sgb, rowb, asem, rsem):
        sl = pl.ds(j * CHUNK, CHUNK)
        pltpu.async_copy(asrc_hbm.at[src_v.at[sl]], sga, asem)
        pltpu.async_copy(adst_hbm.at[dst_v.at[sl]], sgb, asem)
        pltpu.async_copy(h_hbm.at[src_v.at[sl]], rowb, rsem)

    def wait_scatters(idxb, sga, rowb, ssem, dsem):
        pltpu.make_async_copy(rowb, s_sp.at[idxb.at[0]], ssem).wait()
        pltpu.make_async_copy(sga, den_sp.at[idxb.at[0]], dsem).wait()

    def process(j, sga, sgb, idxb, rowb, asem, rsem, ssem, dsem):
        sl = pl.ds(j * CHUNK, CHUNK)
        pltpu.make_async_copy(asrc_hbm.at[src_v.at[sl]], sga, asem).wait()
        pltpu.make_async_copy(adst_hbm.at[dst_v.at[sl]], sgb, asem).wait()
        for t in range(CHUNK // 16):
            tsl = pl.ds(t * 16, 16)
            x = sga[tsl] + sgb[tsl]
            e = jnp.maximum(x, 0.2 * x)
            sga[tsl] = jnp.exp(e)
            idxb[0, tsl] = dst_v[pl.ds(j * CHUNK + t * 16, 16)]
        pltpu.async_copy(sga, den_sp.at[idxb.at[0]], dsem, add=True)
        pltpu.make_async_copy(h_hbm.at[src_v.at[sl]], rowb, rsem).wait()
        def scale(g, _):
            w16 = sga[pl.ds(g * 16, 16)]
            for k in range(16):
                w = w16[k]
                r = g * 16 + k
                for c8 in range(H // 16):
                    csl = pl.ds(c8 * 16, 16)
                    rowb[r, csl] = rowb[r, csl] * w
            return 0
        lax.fori_loop(0, CHUNK // 16, scale, 0)
        pltpu.async_copy(rowb, s_sp.at[idxb.at[0]], ssem, add=True)

    wait_scatters(idx_a, sga_a, row_a, ssem_a, dsem_a)
    fire(0, sga_a, sgb_a, row_a, asem_a, rsem_a)
    def pair(i, _):
        j0 = 2 * i
        wait_scatters(idx_b, sga_b, row_b, ssem_b, dsem_b)
        fire(j0 + 1, sga_b, sgb_b, row_b, asem_b, rsem_b)
        process(j0, sga_a, sgb_a, idx_a, row_a, asem_a, rsem_a, ssem_a, dsem_a)
        process(j0 + 1, sga_b, sgb_b, idx_b, row_b, asem_b, rsem_b, ssem_b, dsem_b)
        wait_scatters(idx_a, sga_a, row_a, ssem_a, dsem_a)
        fire(j0 + 2, sga_a, sgb_a, row_a, asem_a, rsem_a)
        return 0
    lax.fori_loop(0, (NCHUNK - 1) // 2, pair, 0)
    process(NCHUNK - 1, sga_a, sgb_a, idx_a, row_a, asem_a, rsem_a,
            ssem_a, dsem_a)
    wait_scatters(idx_a, sga_a, row_a, ssem_a, dsem_a)
    wait_scatters(idx_b, sga_b, row_b, ssem_b, dsem_b)

    plsc.subcore_barrier()

    def wb(k, _):
        pltpu.sync_copy(s_sp.at[pl.ds(base + k * CHUNK, CHUNK)], row_a)
        pltpu.sync_copy(row_a, s_out.at[cid, pl.ds(base + k * CHUNK, CHUNK)])
        return 0
    lax.fori_loop(0, ROWS_PT // CHUNK, wb, 0)

    pltpu.sync_copy(den_sp.at[pl.ds(dbase, DEN_PT)], den_stage)
    pltpu.sync_copy(den_stage, den_out.at[cid, 0, pl.ds(dbase, DEN_PT)])


_sc_edge = functools.partial(
    pl.kernel,
    out_type=[
        jax.ShapeDtypeStruct((NC, NPAD, H), jnp.float32),
        jax.ShapeDtypeStruct((NC, 2, DEN_PAD), jnp.float32),
    ],
    mesh=plsc.VectorSubcoreMesh(core_axis_name="c", subcore_axis_name="s",
                                num_cores=NC, num_subcores=NS),
    scratch_types=[
        pltpu.VMEM((EPW,), jnp.int32),
        pltpu.VMEM((EPW,), jnp.int32),
        pltpu.VMEM((8, CHUNK), jnp.int32),
        pltpu.VMEM((8, CHUNK), jnp.int32),
        pltpu.VMEM((CHUNK,), jnp.float32),
        pltpu.VMEM((CHUNK,), jnp.float32),
        pltpu.VMEM((CHUNK,), jnp.float32),
        pltpu.VMEM((CHUNK,), jnp.float32),
        pltpu.VMEM((CHUNK, H), jnp.float32),
        pltpu.VMEM((CHUNK, H), jnp.float32),
        pltpu.VMEM((DEN_PT,), jnp.float32),
        pltpu.VMEM_SHARED((NPAD, H), jnp.float32),
        pltpu.VMEM_SHARED((DEN_PAD,), jnp.float32),
        pltpu.SemaphoreType.DMA,
        pltpu.SemaphoreType.DMA,
        pltpu.SemaphoreType.DMA,
        pltpu.SemaphoreType.DMA,
        pltpu.SemaphoreType.DMA,
        pltpu.SemaphoreType.DMA,
        pltpu.SemaphoreType.DMA,
        pltpu.SemaphoreType.DMA,
    ],
)(_sc_edge_body)


def _tc_out_body(s0_ref, s1_ref, d0_ref, d1_ref, bias_ref, cw_ref, cb_ref,
                 y_ref):
    s = s0_ref[0] + s1_ref[0]
    d = d0_ref[...] + d1_ref[...]
    inv = 1.0 / (d + 1e-16)
    x = s * inv + bias_ref[...]
    x = jnp.maximum(x, 0.0)
    y_ref[...] = jnp.dot(x, cw_ref[...],
                         preferred_element_type=jnp.float32) + cb_ref[...]


def _tc_out(s_parts, d0, d1, bias, cw, cb):
    bm = 1024
    return pl.pallas_call(
        _tc_out_body,
        grid=(NPAD // bm,),
        in_specs=[
            pl.BlockSpec((1, bm, H), lambda i: (0, i, 0)),
            pl.BlockSpec((1, bm, H), lambda i: (1, i, 0)),
            pl.BlockSpec((bm, 1), lambda i: (i, 0)),
            pl.BlockSpec((bm, 1), lambda i: (i, 0)),
            pl.BlockSpec((1, H), lambda i: (0, 0)),
            pl.BlockSpec((H, C), lambda i: (0, 0)),
            pl.BlockSpec((1, C), lambda i: (0, 0)),
        ],
        out_specs=pl.BlockSpec((bm, C), lambda i: (i, 0)),
        out_shape=jax.ShapeDtypeStruct((NPAD, C), jnp.float32),
    )(s_parts, s_parts, d0, d1, bias, cw, cb)


def kernel(X_f, edge_index, W, att_src, att_dst, gat_bias, cls_W, cls_b):
    src = edge_index[0].astype(jnp.int32).reshape(NW, EPW)
    dst = edge_index[1].astype(jnp.int32).reshape(NW, EPW)
    att2 = jnp.concatenate(
        [att_src[:, None], att_dst[:, None], jnp.zeros((H, 6), jnp.float32)],
        axis=1)
    h, a2 = _tc_head(X_f, W, att2)
    a_src = a2[:, 0]
    a_dst = a2[:, 1]
    s_parts, den_parts = _sc_edge(src, dst, a_src, a_dst, h)
    d0 = den_parts[0, 0][:, None]
    d1 = den_parts[1, 0][:, None]
    y = _tc_out(s_parts, d0, d1, gat_bias[None, :], cls_W, cls_b[None, :])
    return y[:N]


# speedup vs baseline: 1.0991x; 1.0991x over previous
import functools

import jax
import jax.numpy as jnp
from jax import lax
from jax.experimental import pallas as pl
from jax.experimental.pallas import tpu as pltpu
from jax.experimental.pallas import tpu_sc as plsc

N = 10000
E = 320000
H = 128
C = 16
NC, NS = 2, 16
NW = NC * NS
EPW = E // NW
CHUNK = 80
NCHUNK = EPW // CHUNK
NPAD = 10240
ROWS_PT = NPAD // NS
DEN_PAD = 10240
DEN_PT = DEN_PAD // NS


def _tc_head_body(x_ref, w_ref, att_ref, h_ref, a_ref):
    h = jnp.dot(x_ref[...], w_ref[...], preferred_element_type=jnp.float32)
    h_ref[...] = h
    a_ref[...] = jnp.dot(h, att_ref[...], preferred_element_type=jnp.float32)


def _tc_head(x, w, att2):
    bm = 1000
    return pl.pallas_call(
        _tc_head_body,
        grid=(N // bm,),
        in_specs=[
            pl.BlockSpec((bm, H), lambda i: (i, 0)),
            pl.BlockSpec((H, H), lambda i: (0, 0)),
            pl.BlockSpec((H, 8), lambda i: (0, 0)),
        ],
        out_specs=[
            pl.BlockSpec((bm, H), lambda i: (i, 0)),
            pl.BlockSpec((bm, 8), lambda i: (i, 0)),
        ],
        out_shape=[
            jax.ShapeDtypeStruct((N, H), jnp.float32),
            jax.ShapeDtypeStruct((N, 8), jnp.float32),
        ],
    )(x, w, att2)


def _sc_edge_body(src_hbm, dst_hbm, asrc_hbm, adst_hbm, h_hbm,
                  s_out, den_out,
                  src_v, dst_v, idx_a, idx_b, sga_a, sgb_a, sga_b, sgb_b,
                  row_a, row_b, den_stage, s_sp, den_sp,
                  asem_a, asem_b, rsem_a, rsem_b):
    cid = lax.axis_index("c")
    sid = lax.axis_index("s")
    wid = cid * NS + sid

    zero16 = jnp.zeros((16,), jnp.float32)

    pltpu.sync_copy(src_hbm.at[wid], src_v)
    pltpu.sync_copy(dst_hbm.at[wid], dst_v)

    def zrow(r, _):
        for c8 in range(H // 16):
            row_a[r, pl.ds(c8 * 16, 16)] = zero16
        return 0
    lax.fori_loop(0, CHUNK, zrow, 0)
    def zden(t, _):
        den_stage[pl.ds(t * 16, 16)] = zero16
        return 0
    lax.fori_loop(0, DEN_PT // 16, zden, 0)

    base = sid * ROWS_PT
    def zs(k, _):
        pltpu.sync_copy(row_a, s_sp.at[pl.ds(base + k * CHUNK, CHUNK)])
        return 0
    lax.fori_loop(0, ROWS_PT // CHUNK, zs, 0)

    dbase = sid * DEN_PT
    pltpu.sync_copy(den_stage, den_sp.at[pl.ds(dbase, DEN_PT)])

    plsc.subcore_barrier()

    def fire(j, sga, sgb, rowb, asem, rsem):
        sl = pl.ds(j * CHUNK, CHUNK)
        pltpu.async_copy(asrc_hbm.at[src_v.at[sl]], sga, asem)
        pltpu.async_copy(adst_hbm.at[dst_v.at[sl]], sgb, asem)
        pltpu.async_copy(h_hbm.at[src_v.at[sl]], rowb, rsem)

    def process(j, sga, sgb, idxb, rowb, asem, rsem):
        sl = pl.ds(j * CHUNK, CHUNK)
        pltpu.make_async_copy(asrc_hbm.at[src_v.at[sl]], sga, asem).wait()
        pltpu.make_async_copy(adst_hbm.at[dst_v.at[sl]], sgb, asem).wait()
        for t in range(CHUNK // 16):
            tsl = pl.ds(t * 16, 16)
            x = sga[tsl] + sgb[tsl]
            e = jnp.maximum(x, 0.2 * x)
            sga[tsl] = jnp.exp(e)
            idxb[0, tsl] = dst_v[pl.ds(j * CHUNK + t * 16, 16)]
        pltpu.sync_copy(sga, den_sp.at[idxb.at[0]], add=True)
        pltpu.make_async_copy(h_hbm.at[src_v.at[sl]], rowb, rsem).wait()
        def scale(g, _):
            w16 = sga[pl.ds(g * 16, 16)]
            for k in range(16):
                w = w16[k]
                r = g * 16 + k
                for c8 in range(H // 16):
                    csl = pl.ds(c8 * 16, 16)
                    rowb[r, csl] = rowb[r, csl] * w
            return 0
        lax.fori_loop(0, CHUNK // 16, scale, 0)
        pltpu.sync_copy(rowb, s_sp.at[idxb.at[0]], add=True)

    fire(0, sga_a, sgb_a, row_a, asem_a, rsem_a)
    def pair(i, _):
        j0 = 2 * i
        fire(j0 + 1, sga_b, sgb_b, row_b, asem_b, rsem_b)
        process(j0, sga_a, sgb_a, idx_a, row_a, asem_a, rsem_a)
        fire(j0 + 2, sga_a, sgb_a, row_a, asem_a, rsem_a)
        process(j0 + 1, sga_b, sgb_b, idx_b, row_b, asem_b, rsem_b)
        return 0
    lax.fori_loop(0, (NCHUNK - 1) // 2, pair, 0)
    process(NCHUNK - 1, sga_a, sgb_a, idx_a, row_a, asem_a, rsem_a)

    plsc.subcore_barrier()

    def wb(k, _):
        pltpu.sync_copy(s_sp.at[pl.ds(base + k * CHUNK, CHUNK)], row_a)
        pltpu.sync_copy(row_a, s_out.at[cid, pl.ds(base + k * CHUNK, CHUNK)])
        return 0
    lax.fori_loop(0, ROWS_PT // CHUNK, wb, 0)

    pltpu.sync_copy(den_sp.at[pl.ds(dbase, DEN_PT)], den_stage)
    pltpu.sync_copy(den_stage, den_out.at[cid, 0, pl.ds(dbase, DEN_PT)])


_sc_edge = functools.partial(
    pl.kernel,
    out_type=[
        jax.ShapeDtypeStruct((NC, NPAD, H), jnp.float32),
        jax.ShapeDtypeStruct((NC, 2, DEN_PAD), jnp.float32),
    ],
    mesh=plsc.VectorSubcoreMesh(core_axis_name="c", subcore_axis_name="s",
                                num_cores=NC, num_subcores=NS),
    scratch_types=[
        pltpu.VMEM((EPW,), jnp.int32),
        pltpu.VMEM((EPW,), jnp.int32),
        pltpu.VMEM((8, CHUNK), jnp.int32),
        pltpu.VMEM((8, CHUNK), jnp.int32),
        pltpu.VMEM((CHUNK,), jnp.float32),
        pltpu.VMEM((CHUNK,), jnp.float32),
        pltpu.VMEM((CHUNK,), jnp.float32),
        pltpu.VMEM((CHUNK,), jnp.float32),
        pltpu.VMEM((CHUNK, H), jnp.float32),
        pltpu.VMEM((CHUNK, H), jnp.float32),
        pltpu.VMEM((DEN_PT,), jnp.float32),
        pltpu.VMEM_SHARED((NPAD, H), jnp.float32),
        pltpu.VMEM_SHARED((DEN_PAD,), jnp.float32),
        pltpu.SemaphoreType.DMA,
        pltpu.SemaphoreType.DMA,
        pltpu.SemaphoreType.DMA,
        pltpu.SemaphoreType.DMA,
    ],
)(_sc_edge_body)


def _tc_out_body(s0_ref, s1_ref, d0_ref, d1_ref, bias_ref, cw_ref, cb_ref,
                 y_ref):
    s = s0_ref[0] + s1_ref[0]
    d = d0_ref[...] + d1_ref[...]
    inv = 1.0 / (d + 1e-16)
    x = s * inv + bias_ref[...]
    x = jnp.maximum(x, 0.0)
    y_ref[...] = jnp.dot(x, cw_ref[...],
                         preferred_element_type=jnp.float32) + cb_ref[...]


def _tc_out(s_parts, d0, d1, bias, cw, cb):
    bm = 1024
    return pl.pallas_call(
        _tc_out_body,
        grid=(NPAD // bm,),
        in_specs=[
            pl.BlockSpec((1, bm, H), lambda i: (0, i, 0)),
            pl.BlockSpec((1, bm, H), lambda i: (1, i, 0)),
            pl.BlockSpec((bm, 1), lambda i: (i, 0)),
            pl.BlockSpec((bm, 1), lambda i: (i, 0)),
            pl.BlockSpec((1, H), lambda i: (0, 0)),
            pl.BlockSpec((H, C), lambda i: (0, 0)),
            pl.BlockSpec((1, C), lambda i: (0, 0)),
        ],
        out_specs=pl.BlockSpec((bm, C), lambda i: (i, 0)),
        out_shape=jax.ShapeDtypeStruct((NPAD, C), jnp.float32),
    )(s_parts, s_parts, d0, d1, bias, cw, cb)


def kernel(X_f, edge_index, W, att_src, att_dst, gat_bias, cls_W, cls_b):
    src = edge_index[0].astype(jnp.int32).reshape(NW, EPW)
    dst = edge_index[1].astype(jnp.int32).reshape(NW, EPW)
    att2 = jnp.concatenate(
        [att_src[:, None], att_dst[:, None], jnp.zeros((H, 6), jnp.float32)],
        axis=1)
    h, a2 = _tc_head(X_f, W, att2)
    a_src = a2[:, 0]
    a_dst = a2[:, 1]
    s_parts, den_parts = _sc_edge(src, dst, a_src, a_dst, h)
    d0 = den_parts[0, 0][:, None]
    d1 = den_parts[1, 0][:, None]
    y = _tc_out(s_parts, d0, d1, gat_bias[None, :], cls_W, cls_b[None, :])
    return y[:N]
